# baseline (speedup 1.0000x reference)
import dataclasses
import functools

import jax
import jax.numpy as jnp
from jax import lax
from jax.experimental import pallas as pl
from jax.experimental.pallas import tpu as pltpu
from jax.experimental.pallas import tpu_sc as plsc

B, N, K = 256, 65536, 10
NB = 8192
GRID = N // NB

NUM_CORES = 2
NUM_SUBCORES = 16
NUM_WORKERS = NUM_CORES * NUM_SUBCORES
CH = N // NUM_WORKERS
LANES = 16


def _sc_prep_body(rates_hbm, w_hbm, idx_hbm, r_v, w_v, idx_v, sem):
    wid = lax.axis_index("s") * NUM_CORES + lax.axis_index("c")
    base = wid * CH
    pltpu.async_copy(rates_hbm.at[pl.ds(base * K, CH * K)], r_v, sem).wait()

    iota10 = jnp.arange(LANES, dtype=jnp.int32) * K

    @pl.loop(0, CH, step=LANES)
    def _(i16):
        flat = i16 * K
        v = plsc.load_gather(r_v, [iota10 + flat])
        l1 = jnp.abs(v)
        m = v
        idx = jnp.zeros((LANES,), jnp.int32)
        for k in range(1, K):
            vk = plsc.load_gather(r_v, [iota10 + (flat + k)])
            l1 = l1 + jnp.abs(vk)
            gt = vk > m
            m = jnp.where(gt, vk, m)
            idx = jnp.where(gt, k, idx)
        w = m / jnp.maximum(l1, 1e-12)
        w_v[pl.ds(i16, LANES)] = w
        idx_v[pl.ds(i16, LANES)] = idx

    pltpu.async_copy(w_v, w_hbm.at[pl.ds(base, CH)], sem).wait()
    pltpu.async_copy(idx_v, idx_hbm.at[pl.ds(base, CH)], sem).wait()


def _sc_prep(rates_flat):
    mesh = plsc.VectorSubcoreMesh(core_axis_name="c", subcore_axis_name="s")
    cp = pltpu.CompilerParams()
    if "needs_layout_passes" in pltpu.CompilerParams.__dataclass_fields__:
        cp = dataclasses.replace(cp, needs_layout_passes=False)
    return pl.kernel(
        _sc_prep_body,
        out_type=[
            jax.ShapeDtypeStruct((N,), jnp.float32),
            jax.ShapeDtypeStruct((N,), jnp.int32),
        ],
        mesh=mesh,
        scratch_types=[
            pltpu.VMEM((CH * K,), jnp.float32),
            pltpu.VMEM((CH,), jnp.float32),
            pltpu.VMEM((CH,), jnp.int32),
            pltpu.SemaphoreType.DMA,
        ],
        compiler_params=cp,
    )(rates_flat)


def _mrc_kernel(inputs_ref, w_ref, idx_ref, out_ref, acc_ref, occ_ref):
    i = pl.program_id(0)

    wrow = w_ref[...]
    irow = idx_ref[...]
    sub = lax.broadcasted_iota(jnp.int32, (K, NB), 0)
    onehot = sub == irow
    s_t = jnp.where(onehot, wrow, 0.0)

    part = lax.dot_general(
        inputs_ref[...], s_t,
        dimension_numbers=(((1,), (1,)), ((), ())),
        preferred_element_type=jnp.float32,
    )
    occ_part = jnp.sum(onehot.astype(jnp.float32), axis=1, keepdims=True)

    @pl.when(i == 0)
    def _():
        acc_ref[...] = jnp.zeros_like(acc_ref)
        occ_ref[...] = jnp.zeros_like(occ_ref)

    acc_ref[...] += part
    occ_ref[...] += occ_part

    @pl.when(i == GRID - 1)
    def _():
        occ = occ_ref[...].reshape(1, K)
        q = acc_ref[...] / occ
        q = jnp.where(jnp.isnan(q), 0.0, q)
        q = jnp.where(q == jnp.inf, 0.0, q)
        q = jnp.where(q == -jnp.inf, jnp.finfo(jnp.float32).min, q)
        out_ref[...] = q


@jax.jit
def kernel(inputs, rates):
    w, idx = _sc_prep(rates.reshape(N * K))
    return pl.pallas_call(
        _mrc_kernel,
        grid=(GRID,),
        in_specs=[
            pl.BlockSpec((B, NB), lambda i: (0, i)),
            pl.BlockSpec((1, NB), lambda i: (0, i)),
            pl.BlockSpec((1, NB), lambda i: (0, i)),
        ],
        out_specs=pl.BlockSpec((B, K), lambda i: (0, 0)),
        out_shape=jax.ShapeDtypeStruct((B, K), jnp.float32),
        scratch_shapes=[
            pltpu.VMEM((B, K), jnp.float32),
            pltpu.VMEM((K, 1), jnp.float32),
        ],
        compiler_params=pltpu.CompilerParams(
            dimension_semantics=("arbitrary",),
        ),
    )(inputs, w.reshape(1, N), idx.reshape(1, N))


# speedup vs baseline: 1.9673x; 1.9673x over previous
import dataclasses
import functools

import jax
import jax.numpy as jnp
from jax import lax
from jax.experimental import pallas as pl
from jax.experimental.pallas import tpu as pltpu
from jax.experimental.pallas import tpu_sc as plsc

B, N, K = 256, 65536, 10
NB = 8192
GRID = N // NB

NUM_CORES = 2
NUM_SUBCORES = 16
NUM_WORKERS = NUM_CORES * NUM_SUBCORES
CH = N // NUM_WORKERS
LANES = 16


def _sc_prep_body(rates_t_hbm, w_hbm, idx_hbm, r_v, w_v, idx_v, sem):
    wid = lax.axis_index("s") * NUM_CORES + lax.axis_index("c")
    base = wid * CH
    pltpu.async_copy(rates_t_hbm.at[:, pl.ds(base, CH)], r_v, sem).wait()

    @plsc.parallel_loop(0, CH, step=LANES, unroll=4)
    def _(i16):
        v = r_v[0, pl.ds(i16, LANES)]
        l1 = jnp.abs(v)
        m = v
        idx = jnp.zeros((LANES,), jnp.int32)
        for k in range(1, K):
            vk = r_v[k, pl.ds(i16, LANES)]
            l1 = l1 + jnp.abs(vk)
            gt = vk > m
            m = jnp.where(gt, vk, m)
            idx = jnp.where(gt, k, idx)
        w = m / jnp.maximum(l1, 1e-12)
        w_v[pl.ds(i16, LANES)] = w
        idx_v[pl.ds(i16, LANES)] = idx

    pltpu.async_copy(w_v, w_hbm.at[pl.ds(base, CH)], sem).wait()
    pltpu.async_copy(idx_v, idx_hbm.at[pl.ds(base, CH)], sem).wait()


def _sc_prep(rates_flat):
    mesh = plsc.VectorSubcoreMesh(core_axis_name="c", subcore_axis_name="s")
    cp = pltpu.CompilerParams()
    if "needs_layout_passes" in pltpu.CompilerParams.__dataclass_fields__:
        cp = dataclasses.replace(cp, needs_layout_passes=False)
    return pl.kernel(
        _sc_prep_body,
        out_type=[
            jax.ShapeDtypeStruct((N,), jnp.float32),
            jax.ShapeDtypeStruct((N,), jnp.int32),
        ],
        mesh=mesh,
        scratch_types=[
            pltpu.VMEM((K, CH), jnp.float32),
            pltpu.VMEM((CH,), jnp.float32),
            pltpu.VMEM((CH,), jnp.int32),
            pltpu.SemaphoreType.DMA,
        ],
        compiler_params=cp,
    )(rates_flat)


def _mrc_kernel(inputs_ref, w_ref, idx_ref, out_ref, acc_ref, occ_ref):
    i = pl.program_id(0)

    wrow = w_ref[...]
    irow = idx_ref[...]
    sub = lax.broadcasted_iota(jnp.int32, (K, NB), 0)
    onehot = sub == irow
    s_t = jnp.where(onehot, wrow, 0.0)

    part = lax.dot_general(
        inputs_ref[...], s_t,
        dimension_numbers=(((1,), (1,)), ((), ())),
        preferred_element_type=jnp.float32,
    )
    occ_part = jnp.sum(onehot.astype(jnp.float32), axis=1, keepdims=True)

    @pl.when(i == 0)
    def _():
        acc_ref[...] = jnp.zeros_like(acc_ref)
        occ_ref[...] = jnp.zeros_like(occ_ref)

    acc_ref[...] += part
    occ_ref[...] += occ_part

    @pl.when(i == GRID - 1)
    def _():
        occ = occ_ref[...].reshape(1, K)
        q = acc_ref[...] / occ
        q = jnp.where(jnp.isnan(q), 0.0, q)
        q = jnp.where(q == jnp.inf, 0.0, q)
        q = jnp.where(q == -jnp.inf, jnp.finfo(jnp.float32).min, q)
        out_ref[...] = q


@jax.jit
def kernel(inputs, rates):
    w, idx = _sc_prep(rates.T)
    return pl.pallas_call(
        _mrc_kernel,
        grid=(GRID,),
        in_specs=[
            pl.BlockSpec((B, NB), lambda i: (0, i)),
            pl.BlockSpec((1, NB), lambda i: (0, i)),
            pl.BlockSpec((1, NB), lambda i: (0, i)),
        ],
        out_specs=pl.BlockSpec((B, K), lambda i: (0, 0)),
        out_shape=jax.ShapeDtypeStruct((B, K), jnp.float32),
        scratch_shapes=[
            pltpu.VMEM((B, K), jnp.float32),
            pltpu.VMEM((K, 1), jnp.float32),
        ],
        compiler_params=pltpu.CompilerParams(
            dimension_semantics=("arbitrary",),
        ),
    )(inputs, w.reshape(1, N), idx.reshape(1, N))
